# Initial kernel scaffold; baseline (speedup 1.0000x reference)
import jax
import jax.numpy as jnp
from jax.experimental import pallas as pl


def kernel(x, params, consts):
    raise NotImplementedError("write your pallas kernel here")


# speedup vs baseline: 1.0036x; 1.0036x over previous
import functools
import math

import numpy as np
import jax
import jax.numpy as jnp
from jax.experimental import pallas as pl
from jax.experimental.pallas import tpu as pltpu

_K = 20
_NK = 9
_GROUP = 4
_HI = jax.lax.Precision.HIGHEST
_SQRT2 = math.sqrt(2.0)
_TWO_PI = 2.0 * math.pi
_F32 = jnp.float32


def _gelu(v):
    return 0.5 * v * (1.0 + jax.lax.erf(v / _SQRT2))


def _dot(a, b, precision=_HI):
    return jax.lax.dot_general(a, b, (((1,), (0,)), ((), ())),
                               precision=precision,
                               preferred_element_type=_F32)


def _bf16_dot3(a, b):
    bf = lambda v: v.astype(jnp.bfloat16).astype(_F32)
    t = [bf(a[:, dd:dd + 1]) * bf(b[dd:dd + 1, :]) for dd in range(3)]
    s1 = t[0] + t[1]
    v = s1 - t[0]
    e1 = (t[0] - (s1 - v)) + (t[1] - v)
    s2 = s1 + t[2]
    v2 = s2 - s1
    e2 = (s1 - (s2 - v2)) + (t[2] - v2)
    return s2 + (e1 + e2)


def _scores(q, pt):
    dot = _bf16_dot3(q, pt)
    qq = (q[:, 0:1] * q[:, 0:1] + q[:, 1:2] * q[:, 1:2]) + q[:, 2:3] * q[:, 2:3]
    pp = (pt[0:1] * pt[0:1] + pt[1:2] * pt[1:2]) + pt[2:3] * pt[2:3]
    return (2.0 * dot - qq) - pp


def _select_next(s, iota, n):
    m = jnp.max(s, axis=1, keepdims=True)
    sel = jnp.min(jnp.where(s == m, iota, n), axis=1, keepdims=True)
    oh = iota == sel
    return oh.astype(_F32), jnp.where(oh, jnp.float32(-jnp.inf), s)


def _bf(v):
    return v.astype(jnp.bfloat16).astype(_F32)


_DEF = jax.lax.Precision.DEFAULT


def _paiconv_body(ptst_ref, pts_ref, q_ref, feat_ref, bia_ref, bib_ref,
                  bic_ref, mwt_ref, mb_ref, kn_ref, op_ref, wf_ref, wx_ref,
                  sc_ref, bs_ref, out_ref, ftbuf, xbuf, prawbuf, pmbuf,
                  *, n, tq, f, icx, oc):
    pt = ptst_ref[0]
    p = pts_ref[0]
    q = q_ref[0]
    fm = feat_ref[0]
    iota = jax.lax.broadcasted_iota(jnp.int32, (tq, n), 1)
    bia = bia_ref[...]
    bib = bib_ref[...]
    bic = bic_ref[...]
    mwt_s = mwt_ref[:32, :]
    mwt_c = mwt_ref[32:, :]
    mb = mb_ref[...]
    kn = kn_ref[...]

    def sel_body(k, carry):
        s, pos0 = carry
        oh, s = _select_next(s, iota, n)
        pos_k = _dot(oh, p)
        ftbuf[k] = _bf(_dot(oh, fm))
        pos0 = jnp.where(k == 0, pos_k, pos0)
        rel = pos_k - pos0
        dis = jnp.sqrt(jnp.sum(rel * rel, axis=1, keepdims=True) + 1e-12)
        pr = (_dot(pos0 * _TWO_PI, bia, _DEF) + _dot(rel * _TWO_PI, bib, _DEF)
              + _bf(dis * _TWO_PI) * _bf(bic))
        xbuf[k] = _bf(_dot(jnp.sin(pr), mwt_s, _DEF)
                      + _dot(jnp.cos(pr), mwt_c, _DEF) + mb)
        prawbuf[k] = _bf16_dot3(rel, kn) + op_ref[k]
        return s, pos0

    jax.lax.fori_loop(0, _K, sel_body,
                      (_scores(q, pt), jnp.zeros((tq, 3), _F32)))

    praw = prawbuf[...]
    e = jnp.exp(praw - jnp.max(praw, axis=0, keepdims=True))
    v = e / jnp.sum(e, axis=0, keepdims=True)
    pmv = jnp.where(v > 0.1, v, 0.0)
    pmbuf[...] = _bf(pmv / (jnp.sum(pmv, axis=0, keepdims=True) + 1e-6))

    out = jnp.zeros((tq, oc), _F32)
    for j in range(_NK):
        def gacc(k, c, j=j):
            gf, gx = c
            pmkj = pmbuf[k][:, j:j + 1]
            return gf + ftbuf[k] * pmkj, gx + xbuf[k] * pmkj
        gf, gx = jax.lax.fori_loop(
            0, _K, gacc,
            (jnp.zeros((tq, f), _F32), jnp.zeros((tq, icx), _F32)))
        out = out + _dot(gf, wf_ref[j], _DEF) + _dot(gx, wx_ref[j], _DEF)
    out = sc_ref[...] * out + bs_ref[...]
    out_ref[0] = _gelu(out)


def _paiconv(ptst, pts, feat, bi, kernels, one_padding, mlp_w, mlp_b,
             conv_w, conv_b, bn_w, bn_b, out_c, tq):
    b, n, f = feat.shape
    icx = f // 2 if f > 3 else f
    nf = f + icx
    permute = nf > 3 + icx

    w = conv_w.reshape(out_c, nf, _NK)
    if permute:
        idx = np.arange(nf)
        perm = (idx % _GROUP) * (nf // _GROUP) + idx // _GROUP
        w = w[:, np.argsort(perm), :]
    w3 = jnp.transpose(w, (2, 1, 0))
    wf = w3[:, :f, :]
    wx = w3[:, f:, :]
    bia, bib, bic = bi[:3], bi[3:6], bi[6:7]
    mwt = mlp_w.T
    mb = mlp_b.reshape(1, icx)
    sc = bn_w.reshape(1, out_c)
    bs = (bn_w * conv_b + bn_b).reshape(1, out_c)
    op3 = one_padding.reshape(_K, 1, _NK)

    body = functools.partial(_paiconv_body, n=n, tq=tq, f=f, icx=icx,
                             oc=out_c)
    zmap2 = lambda bb, tt: (0, 0)
    zmap3 = lambda bb, tt: (0, 0, 0)
    return pl.pallas_call(
        body,
        grid=(b, n // tq),
        in_specs=[
            pl.BlockSpec((1, 3, n), lambda bb, tt: (bb, 0, 0)),
            pl.BlockSpec((1, n, 3), lambda bb, tt: (bb, 0, 0)),
            pl.BlockSpec((1, tq, 3), lambda bb, tt: (bb, tt, 0)),
            pl.BlockSpec((1, n, f), lambda bb, tt: (bb, 0, 0)),
            pl.BlockSpec((3, 32), zmap2),
            pl.BlockSpec((3, 32), zmap2),
            pl.BlockSpec((1, 32), zmap2),
            pl.BlockSpec((64, icx), zmap2),
            pl.BlockSpec((1, icx), zmap2),
            pl.BlockSpec((3, _NK), zmap2),
            pl.BlockSpec((_K, 1, _NK), zmap3),
            pl.BlockSpec((_NK, f, out_c), zmap3),
            pl.BlockSpec((_NK, icx, out_c), zmap3),
            pl.BlockSpec((1, out_c), zmap2),
            pl.BlockSpec((1, out_c), zmap2),
        ],
        out_specs=pl.BlockSpec((1, tq, out_c), lambda bb, tt: (bb, tt, 0)),
        out_shape=jax.ShapeDtypeStruct((b, n, out_c), _F32),
        scratch_shapes=[
            pltpu.VMEM((_K, tq, f), _F32),
            pltpu.VMEM((_K, tq, icx), _F32),
            pltpu.VMEM((_K, tq, _NK), _F32),
            pltpu.VMEM((_K, tq, _NK), _F32),
        ],
    )(ptst, pts, pts, feat, bia, bib, bic, mwt, mb, kernels, op3, wf, wx,
      sc, bs)


def _pool_body(ptst_ref, pts_ref, feat_ref, outp_ref, outf_ref, *, n, nq, f):
    pt = ptst_ref[0]
    p = pts_ref[0]
    fm = feat_ref[0]
    q = p[:nq, :]
    iota = jax.lax.broadcasted_iota(jnp.int32, (nq, n), 1)
    neg = jnp.float32(-jnp.inf)

    def body(k, carry):
        s, mp, mf = carry
        oh, s = _select_next(s, iota, n)
        mp = jnp.maximum(mp, _dot(oh, p))
        mf = jnp.maximum(mf, _dot(oh, fm))
        return s, mp, mf

    _, mp, mf = jax.lax.fori_loop(
        0, _K, body,
        (_scores(q, pt), jnp.full((nq, 3), neg), jnp.full((nq, f), neg)))
    outp_ref[0] = mp
    outf_ref[0] = mf


def _pool(ptst, pts, feat, nq):
    b, n, f = feat.shape
    body = functools.partial(_pool_body, n=n, nq=nq, f=f)
    return pl.pallas_call(
        body,
        grid=(b,),
        in_specs=[
            pl.BlockSpec((1, 3, n), lambda bb: (bb, 0, 0)),
            pl.BlockSpec((1, n, 3), lambda bb: (bb, 0, 0)),
            pl.BlockSpec((1, n, f), lambda bb: (bb, 0, 0)),
        ],
        out_specs=[
            pl.BlockSpec((1, nq, 3), lambda bb: (bb, 0, 0)),
            pl.BlockSpec((1, nq, f), lambda bb: (bb, 0, 0)),
        ],
        out_shape=[
            jax.ShapeDtypeStruct((b, nq, 3), _F32),
            jax.ShapeDtypeStruct((b, nq, f), _F32),
        ],
    )(ptst, pts, feat)


def _head_body(h_ref, w5t_ref, s5_ref, b5_ref, l1t_ref, s6_ref, b6_ref,
               l2t_ref, l2b_ref, s7_ref, b7_ref, l3t_ref, l3b_ref, out_ref,
               *, bsize, npool):
    h = h_ref[...]
    v = _dot(h, w5t_ref[...], _DEF)
    v = _gelu(s5_ref[...] * v + b5_ref[...])
    v3 = v.reshape(bsize, npool, v.shape[-1])
    vmax = jnp.max(v3, axis=1)
    vmean = jnp.sum(v3, axis=1) / npool
    h2 = jnp.concatenate([vmax, vmean], axis=1)
    a = _gelu(s6_ref[...] * _dot(h2, l1t_ref[...], _DEF) + b6_ref[...])
    c = _gelu(s7_ref[...] * (_dot(a, l2t_ref[...], _DEF) + l2b_ref[...])
              + b7_ref[...])
    out_ref[...] = _dot(c, l3t_ref[...], _DEF) + l3b_ref[...]


def _head(h2d, params, bsize, npool):
    body = functools.partial(_head_body, bsize=bsize, npool=npool)
    nclass = params['lin3_w'].shape[0]
    return pl.pallas_call(
        body,
        out_shape=jax.ShapeDtypeStruct((bsize, nclass), _F32),
    )(h2d,
      params['conv5_w'].T,
      params['bn5_w'].reshape(1, -1), params['bn5_b'].reshape(1, -1),
      params['lin1_w'].T,
      params['bn6_w'].reshape(1, -1), params['bn6_b'].reshape(1, -1),
      params['lin2_w'].T, params['lin2_b'].reshape(1, -1),
      params['bn7_w'].reshape(1, -1), params['bn7_b'].reshape(1, -1),
      params['lin3_w'].T, params['lin3_b'].reshape(1, -1))


def kernel(x, params, consts):
    b = x.shape[0]
    npts = x.shape[2]
    kn = consts['kernels']
    op = consts['one_padding']

    def layer(i, ptst, pts, feat, out_c, tq):
        return _paiconv(ptst, pts, feat, consts['B%d' % i], kn, op,
                        params['mlp_w%d' % i], params['mlp_b%d' % i],
                        params['conv_w%d' % i], params['conv_b%d' % i],
                        params['bn_w%d' % i], params['bn_b%d' % i],
                        out_c, tq)

    ptst = x
    pts = jnp.transpose(x, (0, 2, 1))

    f1 = layer(1, ptst, pts, pts, 64, 256)
    p1x, p1f = _pool(ptst, pts, f1, npts // 4)
    x1 = p1f[:, :npts // 64, :]

    p1xt = jnp.transpose(p1x, (0, 2, 1))
    f2 = layer(2, p1xt, p1x, p1f, 64, npts // 4)
    p2x, p2f = _pool(p1xt, p1x, f2, npts // 16)
    x2 = p2f[:, :npts // 64, :]

    p2xt = jnp.transpose(p2x, (0, 2, 1))
    f3 = layer(3, p2xt, p2x, p2f, 128, npts // 16)
    p3x, p3f = _pool(p2xt, p2x, f3, npts // 32)
    x3 = p3f[:, :npts // 64, :]

    p3xt = jnp.transpose(p3x, (0, 2, 1))
    f4 = layer(4, p3xt, p3x, p3f, 256, npts // 32)
    _, p4f = _pool(p3xt, p3x, f4, npts // 64)

    h = jnp.concatenate([x1, x2, x3, p4f], axis=2)
    npool = npts // 64
    h2d = h.reshape(b * npool, 512)
    return _head(h2d, params, b, npool)
